# baseline (speedup 1.0000x reference)
import functools

import jax
import jax.numpy as jnp
from jax import lax
from jax.experimental import pallas as pl
from jax.experimental.pallas import tpu as pltpu
from jax.experimental.pallas import tpu_sc as plsc

D = 128
N_NODES = 65535
DEPTH = 16
B_PAD = 65536


_NW = 32
_CH = 128
_NCH = B_PAD // (_NW * _CH)


def _sc_gather(feat2d, emb):
    mesh = plsc.VectorSubcoreMesh(core_axis_name="c", subcore_axis_name="s",
                                  num_cores=2)

    @functools.partial(
        pl.kernel,
        mesh=mesh,
        out_type=jax.ShapeDtypeStruct((B_PAD, D), jnp.float32),
        scratch_types=[
            pltpu.VMEM((_NCH, _CH), jnp.int32),
            pltpu.VMEM((2, _CH, D), jnp.float32),
            pltpu.SemaphoreType.DMA,
            pltpu.SemaphoreType.DMA,
        ],
    )
    def k(feat_hbm, emb_hbm, out_hbm, idx_v, rows_v, sem0, sem1):
        wid = lax.axis_index("s") * 2 + lax.axis_index("c")
        pltpu.sync_copy(feat_hbm.at[pl.ds(wid * _NCH, _NCH)], idx_v)
        sems = (sem0, sem1)
        cps = [None, None]
        cps[0] = pltpu.make_async_copy(
            emb_hbm.at[idx_v.at[0]], rows_v.at[0], sems[0])
        cps[0].start()
        for j in range(_NCH):
            cur = j % 2
            nxt = (j + 1) % 2
            if j + 1 < _NCH:
                cps[nxt] = pltpu.make_async_copy(
                    emb_hbm.at[idx_v.at[j + 1]], rows_v.at[nxt], sems[nxt])
                cps[nxt].start()
            cps[cur].wait()
            pltpu.sync_copy(
                rows_v.at[cur],
                out_hbm.at[pl.ds(wid * (_NCH * _CH) + j * _CH, _CH)])

    return k(feat2d, emb)


_BP = 2048
_NS = 32


def _dotT(a, w):
    return lax.dot_general(a, w, (((1,), (1,)), ((), ())),
                           preferred_element_type=jnp.float32)


def _sigmoid(x):
    return 0.5 * jnp.tanh(0.5 * x) + 0.5


def _gates(iou):
    i = _sigmoid(iou[:, :D])
    o = _sigmoid(iou[:, D:2 * D])
    u = jnp.tanh(iou[:, 2 * D:])
    return i, o, u


def _cell(x, hl, hr, cl, cr, wiou, biou, uiou, wf, bf, uf):
    i, o, u = _gates(_dotT(x, wiou) + biou + _dotT(hl + hr, uiou))
    fb = _dotT(x, wf) + bf
    fl = _sigmoid(fb + _dotT(hl, uf))
    fr = _sigmoid(fb + _dotT(hr, uf))
    c_new = i * u + fl * cl + fr * cr
    h_new = o * jnp.tanh(c_new)
    return h_new, c_new


def _cell_pair(x, hp, cp, wiou, biou, wu2, wf, bf, wfblk):
    i, o, u = _gates(_dotT(x, wiou) + biou + _dotT(hp, wu2))
    fb = _dotT(x, wf) + bf
    g = _dotT(hp, wfblk)
    fl = _sigmoid(fb + g[:, :D])
    fr = _sigmoid(fb + g[:, D:])
    c_new = i * u + fl * cp[:, :D] + fr * cp[:, D:]
    h_new = o * jnp.tanh(c_new)
    return h_new, c_new


def _split_pairs(a):
    a3 = a.reshape(a.shape[0] // 2, 2, a.shape[1])
    return a3[:, 0, :], a3[:, 1, :]


def _fin_offset(pid):
    return jnp.where(
        pid < 16, 32767 + pid * _BP,
        jnp.where(pid < 24, 16383 + (pid - 16) * _BP,
                  jnp.where(pid < 28, 8191 + (pid - 24) * _BP,
                            jnp.where(pid < 30, 4095 + (pid - 28) * _BP,
                                      2047))))


def _store_slot(out_h, out_c, par, h, c):
    @pl.when(par == 0)
    def _():
        out_h[0] = h
        out_c[0] = c

    @pl.when(par == 1)
    def _():
        out_h[1] = h
        out_c[1] = c


def _drain_pair(hfin_ref, cfin_ref, out_h, out_c, sems, par, rows):
    pltpu.make_async_copy(out_h.at[0, pl.ds(0, rows)],
                          hfin_ref.at[pl.ds(0, rows)], sems.at[par]).wait()
    pltpu.make_async_copy(out_c.at[0, pl.ds(0, rows)],
                          cfin_ref.at[pl.ds(0, rows)], sems.at[par]).wait()


def _mega_body(x_ref, wiou_ref, biou_ref, uiou_ref, wf_ref, bf_ref, uf_ref,
               wu2_ref, wfblk_ref, hfin_ref, cfin_ref, hc_in, cc_in,
               out_h, out_c, sems):
    pid = pl.program_id(0)
    par = lax.rem(pid, 2)
    oth = lax.rem(pid + 1, 2)

    @pl.when(jnp.logical_and(pid >= 2, pid <= 30))
    def _():
        _drain_pair(hfin_ref, cfin_ref, out_h, out_c, sems, par, _BP)

    @pl.when(pid >= 30)
    def _():
        _drain_pair(hfin_ref, cfin_ref, out_h, out_c, sems, oth, _BP)

    wiou = wiou_ref[...]
    biou = biou_ref[...]
    uiou = uiou_ref[...]
    wf = wf_ref[...]
    bf = bf_ref[...]
    uf = uf_ref[...]

    @pl.when(pid < 16)
    def _():
        i, o, u = _gates(_dotT(x_ref[...], wiou) + biou)
        c = i * u
        _store_slot(out_h, out_c, par, o * jnp.tanh(c), c)

    @pl.when(jnp.logical_and(pid >= 16, pid < 31))
    def _():
        fin_off = _fin_offset(pid)
        child0 = 2 * fin_off + 1
        hcp = pltpu.make_async_copy(
            hfin_ref.at[pl.ds(child0, 2 * _BP)].reshape(_BP, 2 * D),
            hc_in, sems.at[2])
        ccp = pltpu.make_async_copy(
            cfin_ref.at[pl.ds(child0, 2 * _BP)].reshape(_BP, 2 * D),
            cc_in, sems.at[2])
        hcp.start()
        ccp.start()
        hcp.wait()
        ccp.wait()
        h_new, c_new = _cell_pair(x_ref[...], hc_in[...], cc_in[...],
                                  wiou, biou, wu2_ref[...], wf, bf,
                                  wfblk_ref[...])
        _store_slot(out_h, out_c, par, h_new, c_new)

    @pl.when(pid < 31)
    def _():
        fin_off = _fin_offset(pid)
        pltpu.make_async_copy(out_h.at[par],
                              hfin_ref.at[pl.ds(fin_off, _BP)],
                              sems.at[par]).start()
        pltpu.make_async_copy(out_c.at[par],
                              cfin_ref.at[pl.ds(fin_off, _BP)],
                              sems.at[par]).start()

    @pl.when(pid == 31)
    def _():
        hcp = pltpu.make_async_copy(
            hfin_ref.at[pl.ds(2047, _BP)].reshape(_BP // 2, 2 * D),
            hc_in.at[pl.ds(0, _BP // 2)], sems.at[2])
        ccp = pltpu.make_async_copy(
            cfin_ref.at[pl.ds(2047, _BP)].reshape(_BP // 2, 2 * D),
            cc_in.at[pl.ds(0, _BP // 2)], sems.at[2])
        hcp.start()
        ccp.start()
        hcp.wait()
        ccp.wait()
        hc0 = hc_in[pl.ds(0, _BP // 2), :]
        cc0 = cc_in[pl.ds(0, _BP // 2), :]
        hl, hr = hc0[:, :D], hc0[:, D:]
        cl, cr = cc0[:, :D], cc0[:, D:]
        for d in range(10, -1, -1):
            P = 2 ** d
            x = x_ref[pl.ds(P, P), :]
            h_new, c_new = _cell(x, hl, hr, cl, cr,
                                 wiou, biou, uiou, wf, bf, uf)
            out_h[1, pl.ds(P - 1, P), :] = h_new
            out_c[1, pl.ds(P - 1, P), :] = c_new
            if d > 0:
                hl, hr = _split_pairs(h_new)
                cl, cr = _split_pairs(c_new)
        hcp2 = pltpu.make_async_copy(out_h.at[1, pl.ds(0, 2047)],
                                     hfin_ref.at[pl.ds(0, 2047)],
                                     sems.at[1])
        ccp2 = pltpu.make_async_copy(out_c.at[1, pl.ds(0, 2047)],
                                     cfin_ref.at[pl.ds(0, 2047)],
                                     sems.at[1])
        hcp2.start()
        ccp2.start()
        hcp2.wait()
        ccp2.wait()


def _x_index(i):
    b = jnp.where(i < 16, 16 + i,
                  jnp.where(i < 24, i - 8,
                            jnp.where(i < 28, i - 20,
                                      jnp.where(i < 30, i - 26,
                                                jnp.where(i < 31, 1, 0)))))
    return (b, 0)


def kernel(features, node_order, adjacency_list, edge_order, emb,
           W_iou, b_iou, U_iou, W_f, b_f, U_f):
    f32 = jnp.float32
    b_iou2 = b_iou.reshape(1, 3 * D)
    b_f2 = b_f.reshape(1, D)
    feat2d = jnp.concatenate(
        [jnp.zeros((1,), jnp.int32), features.astype(jnp.int32)]
    ).reshape(B_PAD // D, D)

    x_buf = _sc_gather(feat2d, emb)

    W_u2 = jnp.concatenate([U_iou, U_iou], axis=1)
    z = jnp.zeros((D, D), f32)
    Wfblk = jnp.concatenate(
        [jnp.concatenate([U_f, z], axis=1),
         jnp.concatenate([z, U_f], axis=1)], axis=0)

    cst = lambda i: (0, 0)
    h_fin, c_fin = pl.pallas_call(
        _mega_body,
        grid=(_NS,),
        in_specs=[
            pl.BlockSpec((_BP, D), _x_index),
            pl.BlockSpec((3 * D, D), cst),
            pl.BlockSpec((1, 3 * D), cst),
            pl.BlockSpec((3 * D, D), cst),
            pl.BlockSpec((D, D), cst),
            pl.BlockSpec((1, D), cst),
            pl.BlockSpec((D, D), cst),
            pl.BlockSpec((3 * D, 2 * D), cst),
            pl.BlockSpec((2 * D, 2 * D), cst),
        ],
        out_specs=[pl.BlockSpec(memory_space=pl.ANY)] * 2,
        out_shape=[jax.ShapeDtypeStruct((N_NODES, D), f32)] * 2,
        scratch_shapes=[
            pltpu.VMEM((_BP, 2 * D), f32),
            pltpu.VMEM((_BP, 2 * D), f32),
            pltpu.VMEM((2, _BP, D), f32),
            pltpu.VMEM((2, _BP, D), f32),
            pltpu.SemaphoreType.DMA((3,)),
        ],
        compiler_params=pltpu.CompilerParams(
            dimension_semantics=("arbitrary",)),
    )(x_buf, W_iou, b_iou2, U_iou, W_f, b_f2, U_f, W_u2, Wfblk)

    return (h_fin, c_fin)


# speedup vs baseline: 1.0012x; 1.0012x over previous
import functools

import jax
import jax.numpy as jnp
from jax import lax
from jax.experimental import pallas as pl
from jax.experimental.pallas import tpu as pltpu
from jax.experimental.pallas import tpu_sc as plsc

D = 128
N_NODES = 65535
DEPTH = 16
B_PAD = 65536


_NW = 32
_CH = 128
_NCH = B_PAD // (_NW * _CH)


def _sc_gather(feat2d, emb):
    mesh = plsc.VectorSubcoreMesh(core_axis_name="c", subcore_axis_name="s",
                                  num_cores=2)

    @functools.partial(
        pl.kernel,
        mesh=mesh,
        out_type=jax.ShapeDtypeStruct((B_PAD, D), jnp.float32),
        scratch_types=[
            pltpu.VMEM((_NCH, _CH), jnp.int32),
            pltpu.VMEM((2, _CH, D), jnp.float32),
            pltpu.SemaphoreType.DMA,
            pltpu.SemaphoreType.DMA,
        ],
    )
    def k(feat_hbm, emb_hbm, out_hbm, idx_v, rows_v, sem0, sem1):
        wid = lax.axis_index("s") * 2 + lax.axis_index("c")
        pltpu.sync_copy(feat_hbm.at[pl.ds(wid * _NCH, _NCH)], idx_v)
        sems = (sem0, sem1)
        cps = [None, None]
        cps[0] = pltpu.make_async_copy(
            emb_hbm.at[idx_v.at[0]], rows_v.at[0], sems[0])
        cps[0].start()
        for j in range(_NCH):
            cur = j % 2
            nxt = (j + 1) % 2
            if j + 1 < _NCH:
                cps[nxt] = pltpu.make_async_copy(
                    emb_hbm.at[idx_v.at[j + 1]], rows_v.at[nxt], sems[nxt])
                cps[nxt].start()
            cps[cur].wait()
            pltpu.sync_copy(
                rows_v.at[cur],
                out_hbm.at[pl.ds(wid * (_NCH * _CH) + j * _CH, _CH)])

    return k(feat2d, emb)


_BP = 2048
_NS = 32


def _dotT(a, w):
    return lax.dot_general(a.astype(jnp.bfloat16), w.astype(jnp.bfloat16),
                           (((1,), (1,)), ((), ())),
                           preferred_element_type=jnp.float32)


def _sigmoid(x):
    return 0.5 * jnp.tanh(0.5 * x) + 0.5


def _gates(iou):
    i = _sigmoid(iou[:, :D])
    o = _sigmoid(iou[:, D:2 * D])
    u = jnp.tanh(iou[:, 2 * D:])
    return i, o, u


def _cell(x, hl, hr, cl, cr, wiou, biou, uiou, wf, bf, uf):
    i, o, u = _gates(_dotT(x, wiou) + biou + _dotT(hl + hr, uiou))
    fb = _dotT(x, wf) + bf
    fl = _sigmoid(fb + _dotT(hl, uf))
    fr = _sigmoid(fb + _dotT(hr, uf))
    c_new = i * u + fl * cl + fr * cr
    h_new = o * jnp.tanh(c_new)
    return h_new, c_new


def _cell_pair(x, hp, cp, wiou, biou, wu2, wf, bf, wfblk):
    i, o, u = _gates(_dotT(x, wiou) + biou + _dotT(hp, wu2))
    fb = _dotT(x, wf) + bf
    g = _dotT(hp, wfblk)
    fl = _sigmoid(fb + g[:, :D])
    fr = _sigmoid(fb + g[:, D:])
    c_new = i * u + fl * cp[:, :D] + fr * cp[:, D:]
    h_new = o * jnp.tanh(c_new)
    return h_new, c_new


def _split_pairs(a):
    a3 = a.reshape(a.shape[0] // 2, 2, a.shape[1])
    return a3[:, 0, :], a3[:, 1, :]


def _fin_offset(pid):
    return jnp.where(
        pid < 16, 32767 + pid * _BP,
        jnp.where(pid < 24, 16383 + (pid - 16) * _BP,
                  jnp.where(pid < 28, 8191 + (pid - 24) * _BP,
                            jnp.where(pid < 30, 4095 + (pid - 28) * _BP,
                                      2047))))


def _store_slot(out_h, out_c, par, h, c):
    @pl.when(par == 0)
    def _():
        out_h[0] = h
        out_c[0] = c

    @pl.when(par == 1)
    def _():
        out_h[1] = h
        out_c[1] = c


def _drain_pair(hfin_ref, cfin_ref, out_h, out_c, sems, par, rows):
    pltpu.make_async_copy(out_h.at[0, pl.ds(0, rows)],
                          hfin_ref.at[pl.ds(0, rows)], sems.at[par]).wait()
    pltpu.make_async_copy(out_c.at[0, pl.ds(0, rows)],
                          cfin_ref.at[pl.ds(0, rows)], sems.at[par]).wait()


def _mega_body(x_ref, wiou_ref, biou_ref, uiou_ref, wf_ref, bf_ref, uf_ref,
               wu2_ref, wfblk_ref, hfin_ref, cfin_ref, hc_in, cc_in,
               out_h, out_c, sems):
    pid = pl.program_id(0)
    par = lax.rem(pid, 2)
    oth = lax.rem(pid + 1, 2)

    @pl.when(jnp.logical_and(pid >= 2, pid <= 30))
    def _():
        _drain_pair(hfin_ref, cfin_ref, out_h, out_c, sems, par, _BP)

    @pl.when(pid >= 30)
    def _():
        _drain_pair(hfin_ref, cfin_ref, out_h, out_c, sems, oth, _BP)

    wiou = wiou_ref[...]
    biou = biou_ref[...]
    uiou = uiou_ref[...]
    wf = wf_ref[...]
    bf = bf_ref[...]
    uf = uf_ref[...]

    @pl.when(pid < 16)
    def _():
        i, o, u = _gates(_dotT(x_ref[...], wiou) + biou)
        c = i * u
        _store_slot(out_h, out_c, par, o * jnp.tanh(c), c)

    @pl.when(jnp.logical_and(pid >= 16, pid < 31))
    def _():
        fin_off = _fin_offset(pid)
        child0 = 2 * fin_off + 1
        hcp = pltpu.make_async_copy(
            hfin_ref.at[pl.ds(child0, 2 * _BP)].reshape(_BP, 2 * D),
            hc_in, sems.at[2])
        ccp = pltpu.make_async_copy(
            cfin_ref.at[pl.ds(child0, 2 * _BP)].reshape(_BP, 2 * D),
            cc_in, sems.at[2])
        hcp.start()
        ccp.start()
        hcp.wait()
        ccp.wait()
        h_new, c_new = _cell_pair(x_ref[...], hc_in[...], cc_in[...],
                                  wiou, biou, wu2_ref[...], wf, bf,
                                  wfblk_ref[...])
        _store_slot(out_h, out_c, par, h_new, c_new)

    @pl.when(pid < 31)
    def _():
        fin_off = _fin_offset(pid)
        pltpu.make_async_copy(out_h.at[par],
                              hfin_ref.at[pl.ds(fin_off, _BP)],
                              sems.at[par]).start()
        pltpu.make_async_copy(out_c.at[par],
                              cfin_ref.at[pl.ds(fin_off, _BP)],
                              sems.at[par]).start()

    @pl.when(pid == 31)
    def _():
        hcp = pltpu.make_async_copy(
            hfin_ref.at[pl.ds(2047, _BP)].reshape(_BP // 2, 2 * D),
            hc_in.at[pl.ds(0, _BP // 2)], sems.at[2])
        ccp = pltpu.make_async_copy(
            cfin_ref.at[pl.ds(2047, _BP)].reshape(_BP // 2, 2 * D),
            cc_in.at[pl.ds(0, _BP // 2)], sems.at[2])
        hcp.start()
        ccp.start()
        hcp.wait()
        ccp.wait()
        hc0 = hc_in[pl.ds(0, _BP // 2), :]
        cc0 = cc_in[pl.ds(0, _BP // 2), :]
        hl, hr = hc0[:, :D], hc0[:, D:]
        cl, cr = cc0[:, :D], cc0[:, D:]
        for d in range(10, -1, -1):
            P = 2 ** d
            x = x_ref[pl.ds(P, P), :]
            h_new, c_new = _cell(x, hl, hr, cl, cr,
                                 wiou, biou, uiou, wf, bf, uf)
            out_h[1, pl.ds(P - 1, P), :] = h_new
            out_c[1, pl.ds(P - 1, P), :] = c_new
            if d > 0:
                hl, hr = _split_pairs(h_new)
                cl, cr = _split_pairs(c_new)
        hcp2 = pltpu.make_async_copy(out_h.at[1, pl.ds(0, 2047)],
                                     hfin_ref.at[pl.ds(0, 2047)],
                                     sems.at[1])
        ccp2 = pltpu.make_async_copy(out_c.at[1, pl.ds(0, 2047)],
                                     cfin_ref.at[pl.ds(0, 2047)],
                                     sems.at[1])
        hcp2.start()
        ccp2.start()
        hcp2.wait()
        ccp2.wait()


def _x_index(i):
    b = jnp.where(i < 16, 16 + i,
                  jnp.where(i < 24, i - 8,
                            jnp.where(i < 28, i - 20,
                                      jnp.where(i < 30, i - 26,
                                                jnp.where(i < 31, 1, 0)))))
    return (b, 0)


def kernel(features, node_order, adjacency_list, edge_order, emb,
           W_iou, b_iou, U_iou, W_f, b_f, U_f):
    f32 = jnp.float32
    b_iou2 = b_iou.reshape(1, 3 * D)
    b_f2 = b_f.reshape(1, D)
    feat2d = jnp.concatenate(
        [jnp.zeros((1,), jnp.int32), features.astype(jnp.int32)]
    ).reshape(B_PAD // D, D)

    x_buf = _sc_gather(feat2d, emb)

    W_u2 = jnp.concatenate([U_iou, U_iou], axis=1)
    z = jnp.zeros((D, D), f32)
    Wfblk = jnp.concatenate(
        [jnp.concatenate([U_f, z], axis=1),
         jnp.concatenate([z, U_f], axis=1)], axis=0)

    cst = lambda i: (0, 0)
    h_fin, c_fin = pl.pallas_call(
        _mega_body,
        grid=(_NS,),
        in_specs=[
            pl.BlockSpec((_BP, D), _x_index),
            pl.BlockSpec((3 * D, D), cst),
            pl.BlockSpec((1, 3 * D), cst),
            pl.BlockSpec((3 * D, D), cst),
            pl.BlockSpec((D, D), cst),
            pl.BlockSpec((1, D), cst),
            pl.BlockSpec((D, D), cst),
            pl.BlockSpec((3 * D, 2 * D), cst),
            pl.BlockSpec((2 * D, 2 * D), cst),
        ],
        out_specs=[pl.BlockSpec(memory_space=pl.ANY)] * 2,
        out_shape=[jax.ShapeDtypeStruct((N_NODES, D), f32)] * 2,
        scratch_shapes=[
            pltpu.VMEM((_BP, 2 * D), f32),
            pltpu.VMEM((_BP, 2 * D), f32),
            pltpu.VMEM((2, _BP, D), f32),
            pltpu.VMEM((2, _BP, D), f32),
            pltpu.SemaphoreType.DMA((3,)),
        ],
        compiler_params=pltpu.CompilerParams(
            dimension_semantics=("arbitrary",)),
    )(x_buf, W_iou, b_iou2, U_iou, W_f, b_f2, U_f, W_u2, Wfblk)

    return (h_fin, c_fin)


# speedup vs baseline: 1.0327x; 1.0314x over previous
import functools

import jax
import jax.numpy as jnp
from jax import lax
from jax.experimental import pallas as pl
from jax.experimental.pallas import tpu as pltpu
from jax.experimental.pallas import tpu_sc as plsc

D = 128
N_NODES = 65535
DEPTH = 16
B_PAD = 65536


_NW = 32
_CH = 128
_NCH = B_PAD // (_NW * _CH)


def _sc_gather(feat2d, emb):
    mesh = plsc.VectorSubcoreMesh(core_axis_name="c", subcore_axis_name="s",
                                  num_cores=2)

    @functools.partial(
        pl.kernel,
        mesh=mesh,
        out_type=jax.ShapeDtypeStruct((B_PAD, D), jnp.float32),
        scratch_types=[
            pltpu.VMEM((_NCH, _CH), jnp.int32),
            pltpu.VMEM((2, _CH, D), jnp.float32),
            pltpu.SemaphoreType.DMA,
            pltpu.SemaphoreType.DMA,
        ],
    )
    def k(feat_hbm, emb_hbm, out_hbm, idx_v, rows_v, sem0, sem1):
        wid = lax.axis_index("s") * 2 + lax.axis_index("c")
        pltpu.sync_copy(feat_hbm.at[pl.ds(wid * _NCH, _NCH)], idx_v)
        sems = (sem0, sem1)
        cps = [None, None]
        cps[0] = pltpu.make_async_copy(
            emb_hbm.at[idx_v.at[0]], rows_v.at[0], sems[0])
        cps[0].start()
        for j in range(_NCH):
            cur = j % 2
            nxt = (j + 1) % 2
            if j + 1 < _NCH:
                cps[nxt] = pltpu.make_async_copy(
                    emb_hbm.at[idx_v.at[j + 1]], rows_v.at[nxt], sems[nxt])
                cps[nxt].start()
            cps[cur].wait()
            pltpu.sync_copy(
                rows_v.at[cur],
                out_hbm.at[pl.ds(wid * (_NCH * _CH) + j * _CH, _CH)])

    return k(feat2d, emb)


_BP = 2048
_NS = 32


def _dotT(a, w):
    return lax.dot_general(a, w, (((1,), (1,)), ((), ())),
                           preferred_element_type=jnp.float32)


def _sigmoid(x):
    return 0.5 * jnp.tanh(0.5 * x) + 0.5


def _gates(iou):
    i = _sigmoid(iou[:, :D])
    o = _sigmoid(iou[:, D:2 * D])
    u = jnp.tanh(iou[:, 2 * D:])
    return i, o, u


def _cell(x, hl, hr, cl, cr, wiou, biou, uiou, wf, bf, uf):
    i, o, u = _gates(_dotT(x, wiou) + biou + _dotT(hl + hr, uiou))
    fb = _dotT(x, wf) + bf
    fl = _sigmoid(fb + _dotT(hl, uf))
    fr = _sigmoid(fb + _dotT(hr, uf))
    c_new = i * u + fl * cl + fr * cr
    h_new = o * jnp.tanh(c_new)
    return h_new, c_new


def _cell_pair(x, hp, cp, wiou, biou, wu2, wf, bf, wfblk):
    i, o, u = _gates(_dotT(x, wiou) + biou + _dotT(hp, wu2))
    fb = _dotT(x, wf) + bf
    g = _dotT(hp, wfblk)
    fl = _sigmoid(fb + g[:, :D])
    fr = _sigmoid(fb + g[:, D:])
    c_new = i * u + fl * cp[:, :D] + fr * cp[:, D:]
    h_new = o * jnp.tanh(c_new)
    return h_new, c_new


def _split_pairs(a):
    a3 = a.reshape(a.shape[0] // 2, 2, a.shape[1])
    return a3[:, 0, :], a3[:, 1, :]


def _fin_offset(pid):
    return jnp.where(
        pid < 16, 32767 + pid * _BP,
        jnp.where(pid < 24, 16383 + (pid - 16) * _BP,
                  jnp.where(pid < 28, 8191 + (pid - 24) * _BP,
                            jnp.where(pid < 30, 4095 + (pid - 28) * _BP,
                                      2047))))


def _store_slot(out_h, out_c, par, h, c):
    @pl.when(par == 0)
    def _():
        out_h[0] = h
        out_c[0] = c

    @pl.when(par == 1)
    def _():
        out_h[1] = h
        out_c[1] = c


def _drain_pair(hfin_ref, cfin_ref, out_h, out_c, sems, par, rows):
    pltpu.make_async_copy(out_h.at[0, pl.ds(0, rows)],
                          hfin_ref.at[pl.ds(0, rows)], sems.at[par]).wait()
    pltpu.make_async_copy(out_c.at[0, pl.ds(0, rows)],
                          cfin_ref.at[pl.ds(0, rows)], sems.at[par]).wait()


def _mega_body(x_ref, wiou_ref, biou_ref, uiou_ref, wf_ref, bf_ref, uf_ref,
               wu2_ref, wfblk_ref, hfin_ref, cfin_ref, hc_in, cc_in,
               out_h, out_c, sems):
    pid = pl.program_id(0)
    par = lax.rem(pid, 2)
    oth = lax.rem(pid + 1, 2)

    @pl.when(jnp.logical_and(pid >= 2, pid <= 30))
    def _():
        _drain_pair(hfin_ref, cfin_ref, out_h, out_c, sems, par, _BP)

    @pl.when(pid >= 30)
    def _():
        _drain_pair(hfin_ref, cfin_ref, out_h, out_c, sems, oth, _BP)

    wiou = wiou_ref[...]
    biou = biou_ref[...]
    uiou = uiou_ref[...]
    wf = wf_ref[...]
    bf = bf_ref[...]
    uf = uf_ref[...]

    @pl.when(pid < 16)
    def _():
        i, o, u = _gates(_dotT(x_ref[...], wiou) + biou)
        c = i * u
        _store_slot(out_h, out_c, par, o * jnp.tanh(c), c)

    @pl.when(jnp.logical_and(pid >= 16, pid < 31))
    def _():
        fin_off = _fin_offset(pid)
        child0 = 2 * fin_off + 1

        @pl.when(pid <= 28)
        def _():
            pltpu.make_async_copy(
                hfin_ref.at[pl.ds(child0, 2 * _BP)].reshape(_BP, 2 * D),
                hc_in.at[par], sems.at[3]).wait()
            pltpu.make_async_copy(
                cfin_ref.at[pl.ds(child0, 2 * _BP)].reshape(_BP, 2 * D),
                cc_in.at[par], sems.at[3]).wait()

        @pl.when(pid > 28)
        def _():
            hcp = pltpu.make_async_copy(
                hfin_ref.at[pl.ds(child0, 2 * _BP)].reshape(_BP, 2 * D),
                hc_in.at[par], sems.at[2])
            ccp = pltpu.make_async_copy(
                cfin_ref.at[pl.ds(child0, 2 * _BP)].reshape(_BP, 2 * D),
                cc_in.at[par], sems.at[2])
            hcp.start()
            ccp.start()
            hcp.wait()
            ccp.wait()

        h_new, c_new = _cell_pair(x_ref[...], hc_in[par], cc_in[par],
                                  wiou, biou, wu2_ref[...], wf, bf,
                                  wfblk_ref[...])
        _store_slot(out_h, out_c, par, h_new, c_new)

    @pl.when(pid < 31)
    def _():
        fin_off = _fin_offset(pid)
        pltpu.make_async_copy(out_h.at[par],
                              hfin_ref.at[pl.ds(fin_off, _BP)],
                              sems.at[par]).start()
        pltpu.make_async_copy(out_c.at[par],
                              cfin_ref.at[pl.ds(fin_off, _BP)],
                              sems.at[par]).start()

    @pl.when(jnp.logical_and(pid >= 15, pid <= 27))
    def _():
        nfin = _fin_offset(pid + 1)
        nchild0 = 2 * nfin + 1
        pltpu.make_async_copy(
            hfin_ref.at[pl.ds(nchild0, 2 * _BP)].reshape(_BP, 2 * D),
            hc_in.at[oth], sems.at[3]).start()
        pltpu.make_async_copy(
            cfin_ref.at[pl.ds(nchild0, 2 * _BP)].reshape(_BP, 2 * D),
            cc_in.at[oth], sems.at[3]).start()

    @pl.when(pid == 31)
    def _():
        hcp = pltpu.make_async_copy(
            hfin_ref.at[pl.ds(2047, _BP)].reshape(_BP // 2, 2 * D),
            hc_in.at[1, pl.ds(0, _BP // 2)], sems.at[2])
        ccp = pltpu.make_async_copy(
            cfin_ref.at[pl.ds(2047, _BP)].reshape(_BP // 2, 2 * D),
            cc_in.at[1, pl.ds(0, _BP // 2)], sems.at[2])
        hcp.start()
        ccp.start()
        hcp.wait()
        ccp.wait()
        hc0 = hc_in[1, pl.ds(0, _BP // 2), :]
        cc0 = cc_in[1, pl.ds(0, _BP // 2), :]
        hl, hr = hc0[:, :D], hc0[:, D:]
        cl, cr = cc0[:, :D], cc0[:, D:]
        for d in range(10, -1, -1):
            P = 2 ** d
            x = x_ref[pl.ds(P, P), :]
            h_new, c_new = _cell(x, hl, hr, cl, cr,
                                 wiou, biou, uiou, wf, bf, uf)
            out_h[1, pl.ds(P - 1, P), :] = h_new
            out_c[1, pl.ds(P - 1, P), :] = c_new
            if d > 0:
                hl, hr = _split_pairs(h_new)
                cl, cr = _split_pairs(c_new)
        hcp2 = pltpu.make_async_copy(out_h.at[1, pl.ds(0, 2047)],
                                     hfin_ref.at[pl.ds(0, 2047)],
                                     sems.at[1])
        ccp2 = pltpu.make_async_copy(out_c.at[1, pl.ds(0, 2047)],
                                     cfin_ref.at[pl.ds(0, 2047)],
                                     sems.at[1])
        hcp2.start()
        ccp2.start()
        hcp2.wait()
        ccp2.wait()


def _x_index(i):
    b = jnp.where(i < 16, 16 + i,
                  jnp.where(i < 24, i - 8,
                            jnp.where(i < 28, i - 20,
                                      jnp.where(i < 30, i - 26,
                                                jnp.where(i < 31, 1, 0)))))
    return (b, 0)


def kernel(features, node_order, adjacency_list, edge_order, emb,
           W_iou, b_iou, U_iou, W_f, b_f, U_f):
    f32 = jnp.float32
    b_iou2 = b_iou.reshape(1, 3 * D)
    b_f2 = b_f.reshape(1, D)
    feat2d = jnp.concatenate(
        [jnp.zeros((1,), jnp.int32), features.astype(jnp.int32)]
    ).reshape(B_PAD // D, D)

    x_buf = _sc_gather(feat2d, emb)

    W_u2 = jnp.concatenate([U_iou, U_iou], axis=1)
    z = jnp.zeros((D, D), f32)
    Wfblk = jnp.concatenate(
        [jnp.concatenate([U_f, z], axis=1),
         jnp.concatenate([z, U_f], axis=1)], axis=0)

    cst = lambda i: (0, 0)
    h_fin, c_fin = pl.pallas_call(
        _mega_body,
        grid=(_NS,),
        in_specs=[
            pl.BlockSpec((_BP, D), _x_index),
            pl.BlockSpec((3 * D, D), cst),
            pl.BlockSpec((1, 3 * D), cst),
            pl.BlockSpec((3 * D, D), cst),
            pl.BlockSpec((D, D), cst),
            pl.BlockSpec((1, D), cst),
            pl.BlockSpec((D, D), cst),
            pl.BlockSpec((3 * D, 2 * D), cst),
            pl.BlockSpec((2 * D, 2 * D), cst),
        ],
        out_specs=[pl.BlockSpec(memory_space=pl.ANY)] * 2,
        out_shape=[jax.ShapeDtypeStruct((N_NODES, D), f32)] * 2,
        scratch_shapes=[
            pltpu.VMEM((2, _BP, 2 * D), f32),
            pltpu.VMEM((2, _BP, 2 * D), f32),
            pltpu.VMEM((2, _BP, D), f32),
            pltpu.VMEM((2, _BP, D), f32),
            pltpu.SemaphoreType.DMA((4,)),
        ],
        compiler_params=pltpu.CompilerParams(
            dimension_semantics=("arbitrary",)),
    )(x_buf, W_iou, b_iou2, U_iou, W_f, b_f2, U_f, W_u2, Wfblk)

    return (h_fin, c_fin)


# speedup vs baseline: 1.1286x; 1.0929x over previous
import functools

import jax
import jax.numpy as jnp
from jax import lax
from jax.experimental import pallas as pl
from jax.experimental.pallas import tpu as pltpu
from jax.experimental.pallas import tpu_sc as plsc

D = 128
N_NODES = 65535
DEPTH = 16
B_PAD = 65536


_NW = 32
_CH = 128
_NCH = B_PAD // (_NW * _CH)


def _sc_gather(feat2d, emb):
    mesh = plsc.VectorSubcoreMesh(core_axis_name="c", subcore_axis_name="s",
                                  num_cores=2)

    @functools.partial(
        pl.kernel,
        mesh=mesh,
        out_type=jax.ShapeDtypeStruct((B_PAD, D), jnp.float32),
        scratch_types=[
            pltpu.VMEM((_NCH, _CH), jnp.int32),
            pltpu.VMEM((2, _CH, D), jnp.float32),
            pltpu.SemaphoreType.DMA,
            pltpu.SemaphoreType.DMA,
        ],
    )
    def k(feat_hbm, emb_hbm, out_hbm, idx_v, rows_v, sem0, sem1):
        wid = lax.axis_index("s") * 2 + lax.axis_index("c")
        pltpu.sync_copy(feat_hbm.at[pl.ds(wid * _NCH, _NCH)], idx_v)
        sems = (sem0, sem1)
        cps = [None, None]
        cps[0] = pltpu.make_async_copy(
            emb_hbm.at[idx_v.at[0]], rows_v.at[0], sems[0])
        cps[0].start()
        for j in range(_NCH):
            cur = j % 2
            nxt = (j + 1) % 2
            if j + 1 < _NCH:
                cps[nxt] = pltpu.make_async_copy(
                    emb_hbm.at[idx_v.at[j + 1]], rows_v.at[nxt], sems[nxt])
                cps[nxt].start()
            cps[cur].wait()
            pltpu.sync_copy(
                rows_v.at[cur],
                out_hbm.at[pl.ds(wid * (_NCH * _CH) + j * _CH, _CH)])

    return k(feat2d, emb)


_BP = 4096
_NS = 16


def _dotT(a, w):
    return lax.dot_general(a, w, (((1,), (1,)), ((), ())),
                           preferred_element_type=jnp.float32)


def _sigmoid(x):
    return 0.5 * jnp.tanh(0.5 * x) + 0.5


def _gates(iou):
    i = _sigmoid(iou[:, :D])
    o = _sigmoid(iou[:, D:2 * D])
    u = jnp.tanh(iou[:, 2 * D:])
    return i, o, u


def _cell(x, hl, hr, cl, cr, wiou, biou, uiou, wf, bf, uf):
    i, o, u = _gates(_dotT(x, wiou) + biou + _dotT(hl + hr, uiou))
    fb = _dotT(x, wf) + bf
    fl = _sigmoid(fb + _dotT(hl, uf))
    fr = _sigmoid(fb + _dotT(hr, uf))
    c_new = i * u + fl * cl + fr * cr
    h_new = o * jnp.tanh(c_new)
    return h_new, c_new


def _cell_pair(x, hp, cp, wiou, biou, wu2, wf, bf, wfblk):
    i, o, u = _gates(_dotT(x, wiou) + biou + _dotT(hp, wu2))
    fb = _dotT(x, wf) + bf
    g = _dotT(hp, wfblk)
    fl = _sigmoid(fb + g[:, :D])
    fr = _sigmoid(fb + g[:, D:])
    c_new = i * u + fl * cp[:, :D] + fr * cp[:, D:]
    h_new = o * jnp.tanh(c_new)
    return h_new, c_new


def _split_pairs(a):
    a3 = a.reshape(a.shape[0] // 2, 2, a.shape[1])
    return a3[:, 0, :], a3[:, 1, :]


def _fin_offset(pid):
    return jnp.where(
        pid < 8, 32767 + pid * _BP,
        jnp.where(pid < 12, 16383 + (pid - 8) * _BP,
                  jnp.where(pid < 14, 8191 + (pid - 12) * _BP,
                            4095)))


def _store_slot(out_h, out_c, par, h, c):
    @pl.when(par == 0)
    def _():
        out_h[0] = h
        out_c[0] = c

    @pl.when(par == 1)
    def _():
        out_h[1] = h
        out_c[1] = c


def _drain_pair(hfin_ref, cfin_ref, out_h, out_c, sems, par, rows):
    pltpu.make_async_copy(out_h.at[0, pl.ds(0, rows)],
                          hfin_ref.at[pl.ds(0, rows)], sems.at[par]).wait()
    pltpu.make_async_copy(out_c.at[0, pl.ds(0, rows)],
                          cfin_ref.at[pl.ds(0, rows)], sems.at[par]).wait()


def _mega_body(x_ref, wiou_ref, biou_ref, uiou_ref, wf_ref, bf_ref, uf_ref,
               wu2_ref, wfblk_ref, hfin_ref, cfin_ref, hc_in, cc_in,
               out_h, out_c, sems):
    pid = pl.program_id(0)
    par = lax.rem(pid, 2)
    oth = lax.rem(pid + 1, 2)

    @pl.when(jnp.logical_and(pid >= 2, pid <= 14))
    def _():
        _drain_pair(hfin_ref, cfin_ref, out_h, out_c, sems, par, _BP)

    @pl.when(pid >= 14)
    def _():
        _drain_pair(hfin_ref, cfin_ref, out_h, out_c, sems, oth, _BP)

    wiou = wiou_ref[...]
    biou = biou_ref[...]
    uiou = uiou_ref[...]
    wf = wf_ref[...]
    bf = bf_ref[...]
    uf = uf_ref[...]

    @pl.when(pid < 8)
    def _():
        i, o, u = _gates(_dotT(x_ref[...], wiou) + biou)
        c = i * u
        _store_slot(out_h, out_c, par, o * jnp.tanh(c), c)

    @pl.when(jnp.logical_and(pid >= 8, pid < 15))
    def _():
        fin_off = _fin_offset(pid)
        child0 = 2 * fin_off + 1

        @pl.when(pid <= 12)
        def _():
            pltpu.make_async_copy(
                hfin_ref.at[pl.ds(child0, 2 * _BP)].reshape(_BP, 2 * D),
                hc_in.at[par], sems.at[3]).wait()
            pltpu.make_async_copy(
                cfin_ref.at[pl.ds(child0, 2 * _BP)].reshape(_BP, 2 * D),
                cc_in.at[par], sems.at[3]).wait()

        @pl.when(pid > 12)
        def _():
            hcp = pltpu.make_async_copy(
                hfin_ref.at[pl.ds(child0, 2 * _BP)].reshape(_BP, 2 * D),
                hc_in.at[par], sems.at[2])
            ccp = pltpu.make_async_copy(
                cfin_ref.at[pl.ds(child0, 2 * _BP)].reshape(_BP, 2 * D),
                cc_in.at[par], sems.at[2])
            hcp.start()
            ccp.start()
            hcp.wait()
            ccp.wait()

        h_new, c_new = _cell_pair(x_ref[...], hc_in[par], cc_in[par],
                                  wiou, biou, wu2_ref[...], wf, bf,
                                  wfblk_ref[...])
        _store_slot(out_h, out_c, par, h_new, c_new)

    @pl.when(pid < 15)
    def _():
        fin_off = _fin_offset(pid)
        pltpu.make_async_copy(out_h.at[par],
                              hfin_ref.at[pl.ds(fin_off, _BP)],
                              sems.at[par]).start()
        pltpu.make_async_copy(out_c.at[par],
                              cfin_ref.at[pl.ds(fin_off, _BP)],
                              sems.at[par]).start()

    @pl.when(jnp.logical_and(pid >= 7, pid <= 11))
    def _():
        nfin = _fin_offset(pid + 1)
        nchild0 = 2 * nfin + 1
        pltpu.make_async_copy(
            hfin_ref.at[pl.ds(nchild0, 2 * _BP)].reshape(_BP, 2 * D),
            hc_in.at[oth], sems.at[3]).start()
        pltpu.make_async_copy(
            cfin_ref.at[pl.ds(nchild0, 2 * _BP)].reshape(_BP, 2 * D),
            cc_in.at[oth], sems.at[3]).start()

    @pl.when(pid == 15)
    def _():
        hcp = pltpu.make_async_copy(
            hfin_ref.at[pl.ds(4095, _BP)].reshape(_BP // 2, 2 * D),
            hc_in.at[1, pl.ds(0, _BP // 2)], sems.at[2])
        ccp = pltpu.make_async_copy(
            cfin_ref.at[pl.ds(4095, _BP)].reshape(_BP // 2, 2 * D),
            cc_in.at[1, pl.ds(0, _BP // 2)], sems.at[2])
        hcp.start()
        ccp.start()
        hcp.wait()
        ccp.wait()
        hc0 = hc_in[1, pl.ds(0, _BP // 2), :]
        cc0 = cc_in[1, pl.ds(0, _BP // 2), :]
        hl, hr = hc0[:, :D], hc0[:, D:]
        cl, cr = cc0[:, :D], cc0[:, D:]
        for d in range(11, -1, -1):
            P = 2 ** d
            x = x_ref[pl.ds(P, P), :]
            h_new, c_new = _cell(x, hl, hr, cl, cr,
                                 wiou, biou, uiou, wf, bf, uf)
            out_h[1, pl.ds(P - 1, P), :] = h_new
            out_c[1, pl.ds(P - 1, P), :] = c_new
            if d > 0:
                hl, hr = _split_pairs(h_new)
                cl, cr = _split_pairs(c_new)
        hcp2 = pltpu.make_async_copy(out_h.at[1, pl.ds(0, 4095)],
                                     hfin_ref.at[pl.ds(0, 4095)],
                                     sems.at[1])
        ccp2 = pltpu.make_async_copy(out_c.at[1, pl.ds(0, 4095)],
                                     cfin_ref.at[pl.ds(0, 4095)],
                                     sems.at[1])
        hcp2.start()
        ccp2.start()
        hcp2.wait()
        ccp2.wait()


def _x_index(i):
    b = jnp.where(i < 8, 8 + i,
                  jnp.where(i < 12, i - 4,
                            jnp.where(i < 14, i - 10,
                                      jnp.where(i < 15, 1, 0))))
    return (b, 0)


def kernel(features, node_order, adjacency_list, edge_order, emb,
           W_iou, b_iou, U_iou, W_f, b_f, U_f):
    f32 = jnp.float32
    b_iou2 = b_iou.reshape(1, 3 * D)
    b_f2 = b_f.reshape(1, D)
    feat2d = jnp.concatenate(
        [jnp.zeros((1,), jnp.int32), features.astype(jnp.int32)]
    ).reshape(B_PAD // D, D)

    x_buf = _sc_gather(feat2d, emb)

    W_u2 = jnp.concatenate([U_iou, U_iou], axis=1)
    z = jnp.zeros((D, D), f32)
    Wfblk = jnp.concatenate(
        [jnp.concatenate([U_f, z], axis=1),
         jnp.concatenate([z, U_f], axis=1)], axis=0)

    cst = lambda i: (0, 0)
    h_fin, c_fin = pl.pallas_call(
        _mega_body,
        grid=(_NS,),
        in_specs=[
            pl.BlockSpec((_BP, D), _x_index),
            pl.BlockSpec((3 * D, D), cst),
            pl.BlockSpec((1, 3 * D), cst),
            pl.BlockSpec((3 * D, D), cst),
            pl.BlockSpec((D, D), cst),
            pl.BlockSpec((1, D), cst),
            pl.BlockSpec((D, D), cst),
            pl.BlockSpec((3 * D, 2 * D), cst),
            pl.BlockSpec((2 * D, 2 * D), cst),
        ],
        out_specs=[pl.BlockSpec(memory_space=pl.ANY)] * 2,
        out_shape=[jax.ShapeDtypeStruct((N_NODES, D), f32)] * 2,
        scratch_shapes=[
            pltpu.VMEM((2, _BP, 2 * D), f32),
            pltpu.VMEM((2, _BP, 2 * D), f32),
            pltpu.VMEM((2, _BP, D), f32),
            pltpu.VMEM((2, _BP, D), f32),
            pltpu.SemaphoreType.DMA((4,)),
        ],
        compiler_params=pltpu.CompilerParams(
            dimension_semantics=("arbitrary",)),
    )(x_buf, W_iou, b_iou2, U_iou, W_f, b_f2, U_f, W_u2, Wfblk)

    return (h_fin, c_fin)


# speedup vs baseline: 1.1332x; 1.0040x over previous
import functools

import jax
import jax.numpy as jnp
from jax import lax
from jax.experimental import pallas as pl
from jax.experimental.pallas import tpu as pltpu
from jax.experimental.pallas import tpu_sc as plsc

D = 128
N_NODES = 65535
DEPTH = 16
B_PAD = 65536


_NW = 32
_CH = 128
_NCH = B_PAD // (_NW * _CH)


def _sc_gather(feat2d, emb):
    mesh = plsc.VectorSubcoreMesh(core_axis_name="c", subcore_axis_name="s",
                                  num_cores=2)

    @functools.partial(
        pl.kernel,
        mesh=mesh,
        out_type=jax.ShapeDtypeStruct((B_PAD, D), jnp.float32),
        scratch_types=[
            pltpu.VMEM((_NCH, _CH), jnp.int32),
            pltpu.VMEM((4, _CH, D), jnp.float32),
            pltpu.SemaphoreType.DMA((4,)),
        ],
    )
    def k(feat_hbm, emb_hbm, out_hbm, idx_v, rows_v, sems):
        wid = lax.axis_index("s") * 2 + lax.axis_index("c")
        pltpu.sync_copy(feat_hbm.at[pl.ds(wid * _NCH, _NCH)], idx_v)
        cps = [None] * _NCH
        for j in range(3):
            cps[j] = pltpu.make_async_copy(
                emb_hbm.at[idx_v.at[j]], rows_v.at[j % 4], sems.at[j % 4])
            cps[j].start()
        for j in range(_NCH):
            if j + 3 < _NCH:
                k3 = j + 3
                cps[k3] = pltpu.make_async_copy(
                    emb_hbm.at[idx_v.at[k3]], rows_v.at[k3 % 4],
                    sems.at[k3 % 4])
                cps[k3].start()
            cps[j].wait()
            pltpu.sync_copy(
                rows_v.at[j % 4],
                out_hbm.at[pl.ds(wid * (_NCH * _CH) + j * _CH, _CH)])

    return k(feat2d, emb)


_BP = 4096
_NS = 16


def _dotT(a, w):
    return lax.dot_general(a, w, (((1,), (1,)), ((), ())),
                           preferred_element_type=jnp.float32)


def _sigmoid(x):
    return 0.5 * jnp.tanh(0.5 * x) + 0.5


def _gates(iou):
    i = _sigmoid(iou[:, :D])
    o = _sigmoid(iou[:, D:2 * D])
    u = jnp.tanh(iou[:, 2 * D:])
    return i, o, u


def _cell(x, hl, hr, cl, cr, wiou, biou, uiou, wf, bf, uf):
    i, o, u = _gates(_dotT(x, wiou) + biou + _dotT(hl + hr, uiou))
    fb = _dotT(x, wf) + bf
    fl = _sigmoid(fb + _dotT(hl, uf))
    fr = _sigmoid(fb + _dotT(hr, uf))
    c_new = i * u + fl * cl + fr * cr
    h_new = o * jnp.tanh(c_new)
    return h_new, c_new


def _cell_pair(x, hp, cp, wiou, biou, wu2, wf, bf, wfblk):
    i, o, u = _gates(_dotT(x, wiou) + biou + _dotT(hp, wu2))
    fb = _dotT(x, wf) + bf
    g = _dotT(hp, wfblk)
    fl = _sigmoid(fb + g[:, :D])
    fr = _sigmoid(fb + g[:, D:])
    c_new = i * u + fl * cp[:, :D] + fr * cp[:, D:]
    h_new = o * jnp.tanh(c_new)
    return h_new, c_new


def _split_pairs(a):
    a3 = a.reshape(a.shape[0] // 2, 2, a.shape[1])
    return a3[:, 0, :], a3[:, 1, :]


def _fin_offset(pid):
    return jnp.where(
        pid < 8, 32767 + pid * _BP,
        jnp.where(pid < 12, 16383 + (pid - 8) * _BP,
                  jnp.where(pid < 14, 8191 + (pid - 12) * _BP,
                            4095)))


def _store_slot(out_h, out_c, par, h, c):
    @pl.when(par == 0)
    def _():
        out_h[0] = h
        out_c[0] = c

    @pl.when(par == 1)
    def _():
        out_h[1] = h
        out_c[1] = c


def _drain_pair(hfin_ref, cfin_ref, out_h, out_c, sems, par, rows):
    pltpu.make_async_copy(out_h.at[0, pl.ds(0, rows)],
                          hfin_ref.at[pl.ds(0, rows)], sems.at[par]).wait()
    pltpu.make_async_copy(out_c.at[0, pl.ds(0, rows)],
                          cfin_ref.at[pl.ds(0, rows)], sems.at[par]).wait()


def _mega_body(x_ref, wiou_ref, biou_ref, uiou_ref, wf_ref, bf_ref, uf_ref,
               wu2_ref, wfblk_ref, hfin_ref, cfin_ref, hc_in, cc_in,
               out_h, out_c, sems):
    pid = pl.program_id(0)
    par = lax.rem(pid, 2)
    oth = lax.rem(pid + 1, 2)

    @pl.when(jnp.logical_and(pid >= 2, pid <= 14))
    def _():
        _drain_pair(hfin_ref, cfin_ref, out_h, out_c, sems, par, _BP)

    @pl.when(pid >= 14)
    def _():
        _drain_pair(hfin_ref, cfin_ref, out_h, out_c, sems, oth, _BP)

    wiou = wiou_ref[...]
    biou = biou_ref[...]
    uiou = uiou_ref[...]
    wf = wf_ref[...]
    bf = bf_ref[...]
    uf = uf_ref[...]

    @pl.when(pid < 8)
    def _():
        i, o, u = _gates(_dotT(x_ref[...], wiou) + biou)
        c = i * u
        _store_slot(out_h, out_c, par, o * jnp.tanh(c), c)

    @pl.when(jnp.logical_and(pid >= 8, pid < 15))
    def _():
        fin_off = _fin_offset(pid)
        child0 = 2 * fin_off + 1

        @pl.when(pid <= 12)
        def _():
            pltpu.make_async_copy(
                hfin_ref.at[pl.ds(child0, 2 * _BP)].reshape(_BP, 2 * D),
                hc_in.at[par], sems.at[3]).wait()
            pltpu.make_async_copy(
                cfin_ref.at[pl.ds(child0, 2 * _BP)].reshape(_BP, 2 * D),
                cc_in.at[par], sems.at[3]).wait()

        @pl.when(pid > 12)
        def _():
            hcp = pltpu.make_async_copy(
                hfin_ref.at[pl.ds(child0, 2 * _BP)].reshape(_BP, 2 * D),
                hc_in.at[par], sems.at[2])
            ccp = pltpu.make_async_copy(
                cfin_ref.at[pl.ds(child0, 2 * _BP)].reshape(_BP, 2 * D),
                cc_in.at[par], sems.at[2])
            hcp.start()
            ccp.start()
            hcp.wait()
            ccp.wait()

        h_new, c_new = _cell_pair(x_ref[...], hc_in[par], cc_in[par],
                                  wiou, biou, wu2_ref[...], wf, bf,
                                  wfblk_ref[...])
        _store_slot(out_h, out_c, par, h_new, c_new)

    @pl.when(pid < 15)
    def _():
        fin_off = _fin_offset(pid)
        pltpu.make_async_copy(out_h.at[par],
                              hfin_ref.at[pl.ds(fin_off, _BP)],
                              sems.at[par]).start()
        pltpu.make_async_copy(out_c.at[par],
                              cfin_ref.at[pl.ds(fin_off, _BP)],
                              sems.at[par]).start()

    @pl.when(jnp.logical_and(pid >= 7, pid <= 11))
    def _():
        nfin = _fin_offset(pid + 1)
        nchild0 = 2 * nfin + 1
        pltpu.make_async_copy(
            hfin_ref.at[pl.ds(nchild0, 2 * _BP)].reshape(_BP, 2 * D),
            hc_in.at[oth], sems.at[3]).start()
        pltpu.make_async_copy(
            cfin_ref.at[pl.ds(nchild0, 2 * _BP)].reshape(_BP, 2 * D),
            cc_in.at[oth], sems.at[3]).start()

    @pl.when(pid == 15)
    def _():
        hcp = pltpu.make_async_copy(
            hfin_ref.at[pl.ds(4095, _BP)].reshape(_BP // 2, 2 * D),
            hc_in.at[1, pl.ds(0, _BP // 2)], sems.at[2])
        ccp = pltpu.make_async_copy(
            cfin_ref.at[pl.ds(4095, _BP)].reshape(_BP // 2, 2 * D),
            cc_in.at[1, pl.ds(0, _BP // 2)], sems.at[2])
        hcp.start()
        ccp.start()
        hcp.wait()
        ccp.wait()
        hc0 = hc_in[1, pl.ds(0, _BP // 2), :]
        cc0 = cc_in[1, pl.ds(0, _BP // 2), :]
        hl, hr = hc0[:, :D], hc0[:, D:]
        cl, cr = cc0[:, :D], cc0[:, D:]
        for d in range(11, -1, -1):
            P = 2 ** d
            x = x_ref[pl.ds(P, P), :]
            h_new, c_new = _cell(x, hl, hr, cl, cr,
                                 wiou, biou, uiou, wf, bf, uf)
            out_h[1, pl.ds(P - 1, P), :] = h_new
            out_c[1, pl.ds(P - 1, P), :] = c_new
            if d > 0:
                hl, hr = _split_pairs(h_new)
                cl, cr = _split_pairs(c_new)
        hcp2 = pltpu.make_async_copy(out_h.at[1, pl.ds(0, 4095)],
                                     hfin_ref.at[pl.ds(0, 4095)],
                                     sems.at[1])
        ccp2 = pltpu.make_async_copy(out_c.at[1, pl.ds(0, 4095)],
                                     cfin_ref.at[pl.ds(0, 4095)],
                                     sems.at[1])
        hcp2.start()
        ccp2.start()
        hcp2.wait()
        ccp2.wait()


def _x_index(i):
    b = jnp.where(i < 8, 8 + i,
                  jnp.where(i < 12, i - 4,
                            jnp.where(i < 14, i - 10,
                                      jnp.where(i < 15, 1, 0))))
    return (b, 0)


def kernel(features, node_order, adjacency_list, edge_order, emb,
           W_iou, b_iou, U_iou, W_f, b_f, U_f):
    f32 = jnp.float32
    b_iou2 = b_iou.reshape(1, 3 * D)
    b_f2 = b_f.reshape(1, D)
    feat2d = jnp.concatenate(
        [jnp.zeros((1,), jnp.int32), features.astype(jnp.int32)]
    ).reshape(B_PAD // D, D)

    x_buf = _sc_gather(feat2d, emb)

    W_u2 = jnp.concatenate([U_iou, U_iou], axis=1)
    z = jnp.zeros((D, D), f32)
    Wfblk = jnp.concatenate(
        [jnp.concatenate([U_f, z], axis=1),
         jnp.concatenate([z, U_f], axis=1)], axis=0)

    cst = lambda i: (0, 0)
    h_fin, c_fin = pl.pallas_call(
        _mega_body,
        grid=(_NS,),
        in_specs=[
            pl.BlockSpec((_BP, D), _x_index),
            pl.BlockSpec((3 * D, D), cst),
            pl.BlockSpec((1, 3 * D), cst),
            pl.BlockSpec((3 * D, D), cst),
            pl.BlockSpec((D, D), cst),
            pl.BlockSpec((1, D), cst),
            pl.BlockSpec((D, D), cst),
            pl.BlockSpec((3 * D, 2 * D), cst),
            pl.BlockSpec((2 * D, 2 * D), cst),
        ],
        out_specs=[pl.BlockSpec(memory_space=pl.ANY)] * 2,
        out_shape=[jax.ShapeDtypeStruct((N_NODES, D), f32)] * 2,
        scratch_shapes=[
            pltpu.VMEM((2, _BP, 2 * D), f32),
            pltpu.VMEM((2, _BP, 2 * D), f32),
            pltpu.VMEM((2, _BP, D), f32),
            pltpu.VMEM((2, _BP, D), f32),
            pltpu.SemaphoreType.DMA((4,)),
        ],
        compiler_params=pltpu.CompilerParams(
            dimension_semantics=("arbitrary",)),
    )(x_buf, W_iou, b_iou2, U_iou, W_f, b_f2, U_f, W_u2, Wfblk)

    return (h_fin, c_fin)


# speedup vs baseline: 1.1372x; 1.0035x over previous
import functools

import jax
import jax.numpy as jnp
from jax import lax
from jax.experimental import pallas as pl
from jax.experimental.pallas import tpu as pltpu
from jax.experimental.pallas import tpu_sc as plsc

D = 128
N_NODES = 65535
DEPTH = 16
B_PAD = 65536


_NW = 32
_CH = 128
_NCH = B_PAD // (_NW * _CH)


def _sc_gather(feat2d, emb):
    mesh = plsc.VectorSubcoreMesh(core_axis_name="c", subcore_axis_name="s",
                                  num_cores=2)

    @functools.partial(
        pl.kernel,
        mesh=mesh,
        out_type=jax.ShapeDtypeStruct((B_PAD, D), jnp.float32),
        scratch_types=[
            pltpu.VMEM((_NCH, _CH), jnp.int32),
            pltpu.VMEM((4, _CH, D), jnp.float32),
            pltpu.SemaphoreType.DMA((4,)),
        ],
    )
    def k(feat_hbm, emb_hbm, out_hbm, idx_v, rows_v, sems):
        wid = lax.axis_index("s") * 2 + lax.axis_index("c")
        pltpu.sync_copy(feat_hbm.at[pl.ds(wid * _NCH, _NCH)], idx_v)
        cps = [None] * _NCH
        for j in range(3):
            cps[j] = pltpu.make_async_copy(
                emb_hbm.at[idx_v.at[j]], rows_v.at[j % 4], sems.at[j % 4])
            cps[j].start()
        for j in range(_NCH):
            if j + 3 < _NCH:
                k3 = j + 3
                cps[k3] = pltpu.make_async_copy(
                    emb_hbm.at[idx_v.at[k3]], rows_v.at[k3 % 4],
                    sems.at[k3 % 4])
                cps[k3].start()
            cps[j].wait()
            pltpu.sync_copy(
                rows_v.at[j % 4],
                out_hbm.at[pl.ds(wid * (_NCH * _CH) + j * _CH, _CH)])

    return k(feat2d, emb)


_BP = 4096
_NS = 16


def _dotT(a, w):
    return lax.dot_general(a, w, (((1,), (1,)), ((), ())),
                           preferred_element_type=jnp.float32)


def _sigmoid(x):
    return 0.5 * jnp.tanh(0.5 * x) + 0.5


def _gates(iou):
    i = _sigmoid(iou[:, :D])
    o = _sigmoid(iou[:, D:2 * D])
    u = jnp.tanh(iou[:, 2 * D:])
    return i, o, u


def _cell(x, hl, hr, cl, cr, wiou, biou, uiou, wf, bf, uf):
    i, o, u = _gates(_dotT(x, wiou) + biou + _dotT(hl + hr, uiou))
    fb = _dotT(x, wf) + bf
    fl = _sigmoid(fb + _dotT(hl, uf))
    fr = _sigmoid(fb + _dotT(hr, uf))
    c_new = i * u + fl * cl + fr * cr
    h_new = o * jnp.tanh(c_new)
    return h_new, c_new


def _cell_pair(x, hp, cp, wiou, biou, wu2, wf, bf, wfblk):
    i, o, u = _gates(_dotT(x, wiou) + biou + _dotT(hp, wu2))
    fb = _dotT(x, wf) + bf
    g = _dotT(hp, wfblk)
    fl = _sigmoid(fb + g[:, :D])
    fr = _sigmoid(fb + g[:, D:])
    c_new = i * u + fl * cp[:, :D] + fr * cp[:, D:]
    h_new = o * jnp.tanh(c_new)
    return h_new, c_new


def _split_pairs(a):
    a3 = a.reshape(a.shape[0] // 2, 2, a.shape[1])
    return a3[:, 0, :], a3[:, 1, :]


def _fin_offset(pid):
    return jnp.where(
        pid < 8, 32767 + pid * _BP,
        jnp.where(pid < 12, 16383 + (pid - 8) * _BP,
                  jnp.where(pid < 14, 8191 + (pid - 12) * _BP,
                            4095)))


def _store_slot(out_h, out_c, par, h, c):
    @pl.when(par == 0)
    def _():
        out_h[0] = h
        out_c[0] = c

    @pl.when(par == 1)
    def _():
        out_h[1] = h
        out_c[1] = c


def _drain_pair(hfin_ref, cfin_ref, out_h, out_c, sems, par, rows):
    pltpu.make_async_copy(out_h.at[0, pl.ds(0, rows)],
                          hfin_ref.at[pl.ds(0, rows)], sems.at[par]).wait()
    pltpu.make_async_copy(out_c.at[0, pl.ds(0, rows)],
                          cfin_ref.at[pl.ds(0, rows)], sems.at[par]).wait()


def _mega_body(x_ref, wiou_ref, biou_ref, uiou_ref, wf_ref, bf_ref, uf_ref,
               wu2_ref, wfblk_ref, hfin_ref, cfin_ref, hc_in, cc_in,
               out_h, out_c, sems):
    pid = pl.program_id(0)
    par = lax.rem(pid, 2)
    oth = lax.rem(pid + 1, 2)

    @pl.when(jnp.logical_and(pid >= 2, pid <= 14))
    def _():
        _drain_pair(hfin_ref, cfin_ref, out_h, out_c, sems, par, _BP)

    @pl.when(pid >= 14)
    def _():
        _drain_pair(hfin_ref, cfin_ref, out_h, out_c, sems, oth, _BP)

    wiou = wiou_ref[...]
    biou = biou_ref[...]
    uiou = uiou_ref[...]
    wf = wf_ref[...]
    bf = bf_ref[...]
    uf = uf_ref[...]

    @pl.when(pid < 8)
    def _():
        i, o, u = _gates(_dotT(x_ref[...], wiou) + biou)
        c = i * u
        _store_slot(out_h, out_c, par, o * jnp.tanh(c), c)

    @pl.when(jnp.logical_and(pid >= 8, pid < 15))
    def _():
        fin_off = _fin_offset(pid)
        child0 = 2 * fin_off + 1

        @pl.when(pid <= 12)
        def _():
            pltpu.make_async_copy(
                hfin_ref.at[pl.ds(child0, 2 * _BP)].reshape(_BP, 2 * D),
                hc_in.at[par], sems.at[3]).wait()
            pltpu.make_async_copy(
                cfin_ref.at[pl.ds(child0, 2 * _BP)].reshape(_BP, 2 * D),
                cc_in.at[par], sems.at[3]).wait()

        @pl.when(pid > 12)
        def _():
            hcp = pltpu.make_async_copy(
                hfin_ref.at[pl.ds(child0, 2 * _BP)].reshape(_BP, 2 * D),
                hc_in.at[par], sems.at[2])
            ccp = pltpu.make_async_copy(
                cfin_ref.at[pl.ds(child0, 2 * _BP)].reshape(_BP, 2 * D),
                cc_in.at[par], sems.at[2])
            hcp.start()
            ccp.start()
            hcp.wait()
            ccp.wait()

        h_new, c_new = _cell_pair(x_ref[...], hc_in[par], cc_in[par],
                                  wiou, biou, wu2_ref[...], wf, bf,
                                  wfblk_ref[...])
        _store_slot(out_h, out_c, par, h_new, c_new)

    @pl.when(jnp.logical_and(pid >= 7, pid <= 11))
    def _():
        nfin = _fin_offset(pid + 1)
        nchild0 = 2 * nfin + 1
        pltpu.make_async_copy(
            hfin_ref.at[pl.ds(nchild0, 2 * _BP)].reshape(_BP, 2 * D),
            hc_in.at[oth], sems.at[3]).start()
        pltpu.make_async_copy(
            cfin_ref.at[pl.ds(nchild0, 2 * _BP)].reshape(_BP, 2 * D),
            cc_in.at[oth], sems.at[3]).start()

    @pl.when(pid < 15)
    def _():
        fin_off = _fin_offset(pid)
        pltpu.make_async_copy(out_h.at[par],
                              hfin_ref.at[pl.ds(fin_off, _BP)],
                              sems.at[par]).start()
        pltpu.make_async_copy(out_c.at[par],
                              cfin_ref.at[pl.ds(fin_off, _BP)],
                              sems.at[par]).start()

    @pl.when(pid == 15)
    def _():
        hcp = pltpu.make_async_copy(
            hfin_ref.at[pl.ds(4095, _BP)].reshape(_BP // 2, 2 * D),
            hc_in.at[1, pl.ds(0, _BP // 2)], sems.at[2])
        ccp = pltpu.make_async_copy(
            cfin_ref.at[pl.ds(4095, _BP)].reshape(_BP // 2, 2 * D),
            cc_in.at[1, pl.ds(0, _BP // 2)], sems.at[2])
        hcp.start()
        ccp.start()
        hcp.wait()
        ccp.wait()
        hc0 = hc_in[1, pl.ds(0, _BP // 2), :]
        cc0 = cc_in[1, pl.ds(0, _BP // 2), :]
        hl, hr = hc0[:, :D], hc0[:, D:]
        cl, cr = cc0[:, :D], cc0[:, D:]
        for d in range(11, -1, -1):
            P = 2 ** d
            x = x_ref[pl.ds(P, P), :]
            h_new, c_new = _cell(x, hl, hr, cl, cr,
                                 wiou, biou, uiou, wf, bf, uf)
            out_h[1, pl.ds(P - 1, P), :] = h_new
            out_c[1, pl.ds(P - 1, P), :] = c_new
            if d > 0:
                hl, hr = _split_pairs(h_new)
                cl, cr = _split_pairs(c_new)
        hcp2 = pltpu.make_async_copy(out_h.at[1, pl.ds(0, 4095)],
                                     hfin_ref.at[pl.ds(0, 4095)],
                                     sems.at[1])
        ccp2 = pltpu.make_async_copy(out_c.at[1, pl.ds(0, 4095)],
                                     cfin_ref.at[pl.ds(0, 4095)],
                                     sems.at[1])
        hcp2.start()
        ccp2.start()
        hcp2.wait()
        ccp2.wait()


def _x_index(i):
    b = jnp.where(i < 8, 8 + i,
                  jnp.where(i < 12, i - 4,
                            jnp.where(i < 14, i - 10,
                                      jnp.where(i < 15, 1, 0))))
    return (b, 0)


def kernel(features, node_order, adjacency_list, edge_order, emb,
           W_iou, b_iou, U_iou, W_f, b_f, U_f):
    f32 = jnp.float32
    b_iou2 = b_iou.reshape(1, 3 * D)
    b_f2 = b_f.reshape(1, D)
    feat2d = jnp.concatenate(
        [jnp.zeros((1,), jnp.int32), features.astype(jnp.int32)]
    ).reshape(B_PAD // D, D)

    x_buf = _sc_gather(feat2d, emb)

    W_u2 = jnp.concatenate([U_iou, U_iou], axis=1)
    z = jnp.zeros((D, D), f32)
    Wfblk = jnp.concatenate(
        [jnp.concatenate([U_f, z], axis=1),
         jnp.concatenate([z, U_f], axis=1)], axis=0)

    cst = lambda i: (0, 0)
    h_fin, c_fin = pl.pallas_call(
        _mega_body,
        grid=(_NS,),
        in_specs=[
            pl.BlockSpec((_BP, D), _x_index),
            pl.BlockSpec((3 * D, D), cst),
            pl.BlockSpec((1, 3 * D), cst),
            pl.BlockSpec((3 * D, D), cst),
            pl.BlockSpec((D, D), cst),
            pl.BlockSpec((1, D), cst),
            pl.BlockSpec((D, D), cst),
            pl.BlockSpec((3 * D, 2 * D), cst),
            pl.BlockSpec((2 * D, 2 * D), cst),
        ],
        out_specs=[pl.BlockSpec(memory_space=pl.ANY)] * 2,
        out_shape=[jax.ShapeDtypeStruct((N_NODES, D), f32)] * 2,
        scratch_shapes=[
            pltpu.VMEM((2, _BP, 2 * D), f32),
            pltpu.VMEM((2, _BP, 2 * D), f32),
            pltpu.VMEM((2, _BP, D), f32),
            pltpu.VMEM((2, _BP, D), f32),
            pltpu.SemaphoreType.DMA((4,)),
        ],
        compiler_params=pltpu.CompilerParams(
            dimension_semantics=("arbitrary",)),
    )(x_buf, W_iou, b_iou2, U_iou, W_f, b_f2, U_f, W_u2, Wfblk)

    return (h_fin, c_fin)
